# baseline (speedup 1.0000x reference)
import functools

import jax
import jax.numpy as jnp
from jax import lax
from jax.experimental import pallas as pl
from jax.experimental.pallas import tpu as pltpu
from jax.experimental.pallas import tpu_sc as plsc

C = 1000
L = 16


@functools.lru_cache(maxsize=None)
def _build(N: int):
    info = plsc.get_sparse_core_info()
    NC, NS = info.num_cores, info.num_subcores
    NW = NC * NS
    assert N % (NW * L) == 0
    RPW = N // NW
    RPC = 64 if RPW % 64 == 0 else (32 if RPW % 32 == 0 else L)
    NCHUNK = RPW // RPC
    BUF = RPC * C

    mesh = plsc.VectorSubcoreMesh(core_axis_name="c", subcore_axis_name="s")

    @functools.partial(
        pl.kernel,
        mesh=mesh,
        out_type=jax.ShapeDtypeStruct((N * C,), jnp.float32),
        compiler_params=pltpu.CompilerParams(needs_layout_passes=False),
        scratch_types=[
            pltpu.VMEM((RPW,), jnp.int32),
            pltpu.VMEM((BUF,), jnp.float32),
            pltpu.VMEM((BUF,), jnp.float32),
            pltpu.SemaphoreType.DMA,
            pltpu.SemaphoreType.DMA,
        ],
    )
    def onehot(idx_hbm, out_hbm, idx_v, buf0, buf1, sem0, sem1):
        wid = lax.axis_index("s") * NC + lax.axis_index("c")
        row0 = wid * RPW
        pltpu.sync_copy(idx_hbm.at[pl.ds(row0, RPW)], idx_v)

        zeros = jnp.zeros((L,), jnp.float32)
        ones = jnp.ones((L,), jnp.float32)
        lanes = lax.iota(jnp.int32, L)

        U = 8
        def zbody(i, carry):
            for u in range(U):
                off = (i * U + u) * L
                buf0[pl.ds(off, L)] = zeros
                buf1[pl.ds(off, L)] = zeros
            return carry
        lax.fori_loop(0, BUF // (L * U), zbody, 0)

        bufs = (buf0, buf1)
        sems = (sem0, sem1)

        def set_vals(buf, chunk, val_vec):
            for g in range(RPC // L):
                vals = idx_v[pl.ds(chunk * RPC + g * L, L)]
                offs = (lanes + g * L) * C + vals
                plsc.store_scatter(buf, [offs], val_vec)

        copies = [None] * NCHUNK
        for c in range(NCHUNK):
            b = c % 2
            if c >= 2:
                copies[c - 2].wait()
                set_vals(bufs[b], c - 2, zeros)
            set_vals(bufs[b], c, ones)
            copies[c] = pltpu.async_copy(
                bufs[b], out_hbm.at[pl.ds((row0 + c * RPC) * C, BUF)], sems[b])
        if NCHUNK >= 2:
            copies[NCHUNK - 2].wait()
        copies[NCHUNK - 1].wait()

    return onehot


def kernel(inputs):
    B1, B2 = inputs.shape
    N = B1 * B2
    flat = inputs.reshape(N).astype(jnp.int32)
    out = _build(N)(flat)
    return out.reshape(B1, B2, C)


# speedup vs baseline: 1.6006x; 1.6006x over previous
import jax
import jax.numpy as jnp
from jax import lax
from jax.experimental import pallas as pl

C = 1000
G = 8


def _onehot_body(idx_ref, out_ref):
    idx = idx_ref[...]
    iot = lax.broadcasted_iota(jnp.int32, idx.shape + (C,), idx.ndim)
    out_ref[...] = (idx[..., None] == iot).astype(jnp.float32)


def kernel(inputs):
    B1, B2 = inputs.shape
    return pl.pallas_call(
        _onehot_body,
        grid=(B1 // G,),
        in_specs=[pl.BlockSpec((G, B2), lambda i: (i, 0))],
        out_specs=pl.BlockSpec((G, B2, C), lambda i: (i, 0, 0)),
        out_shape=jax.ShapeDtypeStruct((B1, B2, C), jnp.float32),
    )(inputs.astype(jnp.int32))


# speedup vs baseline: 2.0305x; 1.2686x over previous
import jax
import jax.numpy as jnp
from jax import lax
from jax.experimental import pallas as pl

C = 1000
G = 32


def _onehot_body(idx_ref, out_ref):
    idx = idx_ref[...]
    iot = lax.broadcasted_iota(jnp.int32, idx.shape + (C,), idx.ndim)
    out_ref[...] = (idx[..., None] == iot).astype(jnp.float32)


def kernel(inputs):
    B1, B2 = inputs.shape
    return pl.pallas_call(
        _onehot_body,
        grid=(B1 // G,),
        in_specs=[pl.BlockSpec((G, B2), lambda i: (i, 0))],
        out_specs=pl.BlockSpec((G, B2, C), lambda i: (i, 0, 0)),
        out_shape=jax.ShapeDtypeStruct((B1, B2, C), jnp.float32),
    )(inputs.astype(jnp.int32))
